# baseline (speedup 1.0000x reference)
import functools

import jax
import jax.numpy as jnp
from jax import lax
from jax.experimental import pallas as pl
from jax.experimental.pallas import tpu as pltpu
from jax.experimental.pallas import tpu_sc as plsc

VOCAB = 1000000
EMBED = 32
BATCH = 16384
NEG = 20

NUM_CORES = 2
NUM_SUBCORES = 16
NW = NUM_CORES * NUM_SUBCORES
BPW = BATCH // NW
BC = 16
NBLK = BPW // BC
NROWS_BLK = BC * NEG
NCHUNK = 5
CHUNK = NROWS_BLK // NCHUNK

S = 250112
SB = S // 128
COLB_MAX = (VOCAB + 127) // 128 - 1


def _pack_body(w0, w1, w2, w3, u0, u1, u2, u3, ow, ou):
    ow[...] = jnp.concatenate(
        [w0[...], w1[...], w2[...], w3[...]], axis=0).T
    ou[...] = jnp.concatenate(
        [u0[...], u1[...], u2[...], u3[...]], axis=0).T


def _pack_tables(W_w, W_u):
    wt = W_w.T
    ut = W_u.T

    def in_spec(q):
        return pl.BlockSpec(
            (EMBED, 128),
            lambda i, q=q: (0, jnp.minimum(q * SB + i, COLB_MAX)))

    out_spec = pl.BlockSpec((128, 128), lambda i: (i, 0))
    return pl.pallas_call(
        _pack_body,
        grid=(SB,),
        in_specs=[in_spec(q) for q in range(4)] * 2,
        out_specs=[out_spec, out_spec],
        out_shape=[jax.ShapeDtypeStruct((S, 128), jnp.float32)] * 2,
    )(wt, wt, wt, wt, ut, ut, ut, ut)


def _sc_body(wp_w, wp_u, isup_h, tsup_h, nsup_h, icb_h, tcb_h, ncb_h,
             pos_h, negl_h,
             isup_v, tsup_v, nsup_v, icb_v, tcb_v, ncb_v,
             emb_v, ctx_v, nrow_v, pos_v, negl_v, sem):
    cid = lax.axis_index("c")
    sid = lax.axis_index("s")
    wid = sid * NUM_CORES + cid

    pltpu.sync_copy(isup_h.at[wid], isup_v)
    pltpu.sync_copy(tsup_h.at[wid], tsup_v)
    pltpu.sync_copy(nsup_h.at[wid], nsup_v)
    pltpu.sync_copy(icb_h.at[wid], icb_v)
    pltpu.sync_copy(tcb_h.at[wid], tcb_v)
    pltpu.sync_copy(ncb_h.at[wid], ncb_v)

    iota = lax.iota(jnp.int32, 16)
    nrows0 = iota * NEG

    @pl.loop(0, NBLK)
    def _block(blk):
        descs = [
            pltpu.async_copy(wp_w.at[isup_v.at[blk]], emb_v, sem),
            pltpu.async_copy(wp_u.at[tsup_v.at[blk]], ctx_v, sem),
        ]
        for c in range(NCHUNK):
            descs.append(pltpu.async_copy(
                wp_u.at[nsup_v.at[blk * NCHUNK + c]],
                nrow_v.at[pl.ds(c * CHUNK, CHUNK)], sem))
        for d in descs:
            d.wait()

        cb_e = icb_v[pl.ds(blk * BC, 16)]
        cb_c = tcb_v[pl.ds(blk * BC, 16)]
        cb_n = [ncb_v[pl.ds(k * BPW + blk * BC, 16)] for k in range(NEG)]
        acc_p = jnp.zeros((16,), jnp.float32)
        acc_n = jnp.zeros((16,), jnp.float32)
        for d in range(EMBED):
            e = plsc.load_gather(emb_v, [iota, cb_e + d])
            c = plsc.load_gather(ctx_v, [iota, cb_c + d])
            acc_p = acc_p + e * c
            s = plsc.load_gather(nrow_v, [nrows0, cb_n[0] + d])
            for k in range(1, NEG):
                s = s + plsc.load_gather(nrow_v, [nrows0 + k, cb_n[k] + d])
            acc_n = acc_n + e * s
        pos_v[pl.ds(blk * BC, 16)] = acc_p
        negl_v[pl.ds(blk * BC, 16)] = -acc_n

    pltpu.sync_copy(pos_v, pos_h.at[wid])
    pltpu.sync_copy(negl_v, negl_h.at[wid])


def _make_sc_kernel():
    mesh = plsc.VectorSubcoreMesh(core_axis_name="c", subcore_axis_name="s")
    return pl.kernel(
        _sc_body,
        out_type=(
            jax.ShapeDtypeStruct((NW, BPW), jnp.float32),
            jax.ShapeDtypeStruct((NW, BPW), jnp.float32),
        ),
        mesh=mesh,
        scratch_types=(
            pltpu.VMEM((NBLK, BC), jnp.int32),
            pltpu.VMEM((NBLK, BC), jnp.int32),
            pltpu.VMEM((NBLK * NCHUNK, CHUNK), jnp.int32),
            pltpu.VMEM((BPW,), jnp.int32),
            pltpu.VMEM((BPW,), jnp.int32),
            pltpu.VMEM((BPW * NEG,), jnp.int32),
            pltpu.VMEM((BC, 128), jnp.float32),
            pltpu.VMEM((BC, 128), jnp.float32),
            pltpu.VMEM((NROWS_BLK, 128), jnp.float32),
            pltpu.VMEM((BPW,), jnp.float32),
            pltpu.VMEM((BPW,), jnp.float32),
            pltpu.SemaphoreType.DMA,
        ),
        compiler_params=pltpu.CompilerParams(needs_layout_passes=False),
    )


def _loss_body(pos_ref, negl_ref, out_ref):
    def logsig(x):
        return jnp.minimum(x, 0.0) - jnp.log1p(jnp.exp(-jnp.abs(x)))

    total = jnp.sum(logsig(pos_ref[...])) + jnp.sum(logsig(negl_ref[...]))
    out_ref[0, 0] = -total / BATCH


def _split_idx(v):
    q = v // S
    return (v - q * S).astype(jnp.int32), (q * 32).astype(jnp.int32)


@jax.jit
def kernel(inputs, targets, neg_samples, W_w, W_u):
    wp_w, wp_u = _pack_tables(W_w, W_u)

    isup, icb = _split_idx(inputs.astype(jnp.int32).reshape(BATCH))
    tsup, tcb = _split_idx(targets.astype(jnp.int32).reshape(BATCH))
    nsup, ncb = _split_idx(neg_samples.astype(jnp.int32))

    isup_h = isup.reshape(NW, NBLK, BC)
    tsup_h = tsup.reshape(NW, NBLK, BC)
    nsup_h = nsup.reshape(NW, NBLK * NCHUNK, CHUNK)
    icb_h = icb.reshape(NW, BPW)
    tcb_h = tcb.reshape(NW, BPW)
    ncb_h = ncb.reshape(NW, BPW, NEG).transpose(0, 2, 1).reshape(NW, BPW * NEG)

    pos, negl = _make_sc_kernel()(
        wp_w, wp_u, isup_h, tsup_h, nsup_h, icb_h, tcb_h, ncb_h)

    loss = pl.pallas_call(
        _loss_body,
        out_shape=jax.ShapeDtypeStruct((1, 1), jnp.float32),
        out_specs=pl.BlockSpec(memory_space=pltpu.SMEM),
    )(pos.reshape(128, 128), negl.reshape(128, 128))
    return loss[0, 0]


# speedup vs baseline: 2.4118x; 2.4118x over previous
import functools

import jax
import jax.numpy as jnp
from jax import lax
from jax.experimental import pallas as pl
from jax.experimental.pallas import tpu as pltpu
from jax.experimental.pallas import tpu_sc as plsc

VOCAB = 1000000
EMBED = 32
BATCH = 16384
NEG = 20

NUM_CORES = 2
NUM_SUBCORES = 16
NW = NUM_CORES * NUM_SUBCORES
BPW = BATCH // NW
BC = 16
NBLK = BPW // BC
NROWS_BLK = BC * NEG
NCHUNK = 5
CHUNK = NROWS_BLK // NCHUNK

S = 250880
GW = 1024
SBG = S // GW
CLAMP_J = (VOCAB + GW - 1) // GW - 1


def _pack_body(w0, w1, w2, w3, u0, u1, u2, u3, ow, ou):
    ow[...] = jnp.concatenate(
        [w0[...], w1[...], w2[...], w3[...]], axis=0).T
    ou[...] = jnp.concatenate(
        [u0[...], u1[...], u2[...], u3[...]], axis=0).T


def _pack_tables(W_w, W_u):
    wt = W_w.T
    ut = W_u.T

    def in_spec(q):
        return pl.BlockSpec(
            (EMBED, GW),
            lambda i, q=q: (0, jnp.minimum(q * SBG + i, CLAMP_J)))

    out_spec = pl.BlockSpec((GW, 128), lambda i: (i, 0))
    return pl.pallas_call(
        _pack_body,
        grid=(SBG,),
        in_specs=[in_spec(q) for q in range(4)] * 2,
        out_specs=[out_spec, out_spec],
        out_shape=[jax.ShapeDtypeStruct((S, 128), jnp.float32)] * 2,
    )(wt, wt, wt, wt, ut, ut, ut, ut)


def _sc_body(wp_w, wp_u, isup_h, tsup_h, nsup_h, icb_h, tcb_h, ncb_h,
             pos_h, negl_h,
             isup_v, tsup_v, nsup_v, icb_v, tcb_v, ncb_v,
             emb_v, ctx_v, nrow_v, pos_v, negl_v, sem):
    cid = lax.axis_index("c")
    sid = lax.axis_index("s")
    wid = sid * NUM_CORES + cid

    pltpu.sync_copy(isup_h.at[wid], isup_v)
    pltpu.sync_copy(tsup_h.at[wid], tsup_v)
    pltpu.sync_copy(nsup_h.at[wid], nsup_v)
    pltpu.sync_copy(icb_h.at[wid], icb_v)
    pltpu.sync_copy(tcb_h.at[wid], tcb_v)
    pltpu.sync_copy(ncb_h.at[wid], ncb_v)

    iota = lax.iota(jnp.int32, 16)
    nrows0 = iota * NEG

    @pl.loop(0, NBLK)
    def _block(blk):
        descs = [
            pltpu.async_copy(wp_w.at[isup_v.at[blk]], emb_v, sem),
            pltpu.async_copy(wp_u.at[tsup_v.at[blk]], ctx_v, sem),
        ]
        for c in range(NCHUNK):
            descs.append(pltpu.async_copy(
                wp_u.at[nsup_v.at[blk * NCHUNK + c]],
                nrow_v.at[pl.ds(c * CHUNK, CHUNK)], sem))
        for d in descs:
            d.wait()

        cb_e = icb_v[pl.ds(blk * BC, 16)]
        cb_c = tcb_v[pl.ds(blk * BC, 16)]
        cb_n = [ncb_v[pl.ds(k * BPW + blk * BC, 16)] for k in range(NEG)]
        acc_p = jnp.zeros((16,), jnp.float32)
        acc_n = jnp.zeros((16,), jnp.float32)
        for d in range(EMBED):
            e = plsc.load_gather(emb_v, [iota, cb_e + d])
            c = plsc.load_gather(ctx_v, [iota, cb_c + d])
            acc_p = acc_p + e * c
            s = plsc.load_gather(nrow_v, [nrows0, cb_n[0] + d])
            for k in range(1, NEG):
                s = s + plsc.load_gather(nrow_v, [nrows0 + k, cb_n[k] + d])
            acc_n = acc_n + e * s
        pos_v[pl.ds(blk * BC, 16)] = acc_p
        negl_v[pl.ds(blk * BC, 16)] = -acc_n

    pltpu.sync_copy(pos_v, pos_h.at[wid])
    pltpu.sync_copy(negl_v, negl_h.at[wid])


def _make_sc_kernel():
    mesh = plsc.VectorSubcoreMesh(core_axis_name="c", subcore_axis_name="s")
    return pl.kernel(
        _sc_body,
        out_type=(
            jax.ShapeDtypeStruct((NW, BPW), jnp.float32),
            jax.ShapeDtypeStruct((NW, BPW), jnp.float32),
        ),
        mesh=mesh,
        scratch_types=(
            pltpu.VMEM((NBLK, BC), jnp.int32),
            pltpu.VMEM((NBLK, BC), jnp.int32),
            pltpu.VMEM((NBLK * NCHUNK, CHUNK), jnp.int32),
            pltpu.VMEM((BPW,), jnp.int32),
            pltpu.VMEM((BPW,), jnp.int32),
            pltpu.VMEM((BPW * NEG,), jnp.int32),
            pltpu.VMEM((BC, 128), jnp.float32),
            pltpu.VMEM((BC, 128), jnp.float32),
            pltpu.VMEM((NROWS_BLK, 128), jnp.float32),
            pltpu.VMEM((BPW,), jnp.float32),
            pltpu.VMEM((BPW,), jnp.float32),
            pltpu.SemaphoreType.DMA,
        ),
        compiler_params=pltpu.CompilerParams(needs_layout_passes=False),
    )


def _loss_body(pos_ref, negl_ref, out_ref):
    def logsig(x):
        return jnp.minimum(x, 0.0) - jnp.log1p(jnp.exp(-jnp.abs(x)))

    total = jnp.sum(logsig(pos_ref[...])) + jnp.sum(logsig(negl_ref[...]))
    out_ref[0, 0] = -total / BATCH


def _split_idx(v):
    q = v // S
    return (v - q * S).astype(jnp.int32), (q * 32).astype(jnp.int32)


@jax.jit
def kernel(inputs, targets, neg_samples, W_w, W_u):
    wp_w, wp_u = _pack_tables(W_w, W_u)

    isup, icb = _split_idx(inputs.astype(jnp.int32).reshape(BATCH))
    tsup, tcb = _split_idx(targets.astype(jnp.int32).reshape(BATCH))
    nsup, ncb = _split_idx(neg_samples.astype(jnp.int32))

    isup_h = isup.reshape(NW, NBLK, BC)
    tsup_h = tsup.reshape(NW, NBLK, BC)
    nsup_h = nsup.reshape(NW, NBLK * NCHUNK, CHUNK)
    icb_h = icb.reshape(NW, BPW)
    tcb_h = tcb.reshape(NW, BPW)
    ncb_h = ncb.reshape(NW, BPW, NEG).transpose(0, 2, 1).reshape(NW, BPW * NEG)

    pos, negl = _make_sc_kernel()(
        wp_w, wp_u, isup_h, tsup_h, nsup_h, icb_h, tcb_h, ncb_h)

    loss = pl.pallas_call(
        _loss_body,
        out_shape=jax.ShapeDtypeStruct((1, 1), jnp.float32),
        out_specs=pl.BlockSpec(memory_space=pltpu.SMEM),
    )(pos.reshape(128, 128), negl.reshape(128, 128))
    return loss[0, 0]


# speedup vs baseline: 2.9807x; 1.2359x over previous
import functools

import jax
import jax.numpy as jnp
from jax import lax
from jax.experimental import pallas as pl
from jax.experimental.pallas import tpu as pltpu
from jax.experimental.pallas import tpu_sc as plsc

VOCAB = 1000000
EMBED = 32
BATCH = 16384
NEG = 20

NUM_CORES = 2
NUM_SUBCORES = 16
NW = NUM_CORES * NUM_SUBCORES
BPW = BATCH // NW
BC = 16
NBLK = BPW // BC
NROWS_BLK = BC * NEG
NCHUNK = 5
CHUNK = NROWS_BLK // NCHUNK

S = 250880
GW = 5120
SBG = S // GW
CLAMP_J = (VOCAB + GW - 1) // GW - 1


def _pack_body(w0, w1, w2, w3, u0, u1, u2, u3, ow, ou):
    ow[...] = jnp.concatenate(
        [w0[...], w1[...], w2[...], w3[...]], axis=0).T
    ou[...] = jnp.concatenate(
        [u0[...], u1[...], u2[...], u3[...]], axis=0).T


def _pack_tables(W_w, W_u):
    wt = W_w.T
    ut = W_u.T

    def in_spec(q):
        return pl.BlockSpec(
            (EMBED, GW),
            lambda i, q=q: (0, jnp.minimum(q * SBG + i, CLAMP_J)))

    out_spec = pl.BlockSpec((GW, 128), lambda i: (i, 0))
    return pl.pallas_call(
        _pack_body,
        grid=(SBG,),
        in_specs=[in_spec(q) for q in range(4)] * 2,
        out_specs=[out_spec, out_spec],
        out_shape=[jax.ShapeDtypeStruct((S, 128), jnp.float32)] * 2,
    )(wt, wt, wt, wt, ut, ut, ut, ut)


def _sc_body(wp_w, wp_u, isup_h, tsup_h, nsup_h, icb_h, tcb_h, ncb_h,
             pos_h, negl_h,
             isup_v, tsup_v, nsup_v, icb_v, tcb_v, ncb_v,
             emb_v, ctx_v, nrow_v, pos_v, negl_v, sem):
    cid = lax.axis_index("c")
    sid = lax.axis_index("s")
    wid = sid * NUM_CORES + cid

    pltpu.sync_copy(isup_h.at[wid], isup_v)
    pltpu.sync_copy(tsup_h.at[wid], tsup_v)
    pltpu.sync_copy(nsup_h.at[wid], nsup_v)
    pltpu.sync_copy(icb_h.at[wid], icb_v)
    pltpu.sync_copy(tcb_h.at[wid], tcb_v)
    pltpu.sync_copy(ncb_h.at[wid], ncb_v)

    iota = lax.iota(jnp.int32, 16)
    nrows0 = iota * NEG

    @pl.loop(0, NBLK)
    def _block(blk):
        descs = [
            pltpu.async_copy(wp_w.at[isup_v.at[blk]], emb_v, sem),
            pltpu.async_copy(wp_u.at[tsup_v.at[blk]], ctx_v, sem),
        ]
        for c in range(NCHUNK):
            descs.append(pltpu.async_copy(
                wp_u.at[nsup_v.at[blk * NCHUNK + c]],
                nrow_v.at[pl.ds(c * CHUNK, CHUNK)], sem))
        for d in descs:
            d.wait()

        cb_e = icb_v[pl.ds(blk * BC, 16)]
        cb_c = tcb_v[pl.ds(blk * BC, 16)]
        cb_n = [ncb_v[pl.ds(k * BPW + blk * BC, 16)] for k in range(NEG)]
        acc_p = jnp.zeros((16,), jnp.float32)
        acc_n = jnp.zeros((16,), jnp.float32)
        for d in range(EMBED):
            e = plsc.load_gather(emb_v, [iota, cb_e + d])
            c = plsc.load_gather(ctx_v, [iota, cb_c + d])
            acc_p = acc_p + e * c
            s = plsc.load_gather(nrow_v, [nrows0, cb_n[0] + d])
            for k in range(1, NEG):
                s = s + plsc.load_gather(nrow_v, [nrows0 + k, cb_n[k] + d])
            acc_n = acc_n + e * s
        pos_v[pl.ds(blk * BC, 16)] = acc_p
        negl_v[pl.ds(blk * BC, 16)] = -acc_n

    pltpu.sync_copy(pos_v, pos_h.at[wid])
    pltpu.sync_copy(negl_v, negl_h.at[wid])


def _make_sc_kernel():
    mesh = plsc.VectorSubcoreMesh(core_axis_name="c", subcore_axis_name="s")
    return pl.kernel(
        _sc_body,
        out_type=(
            jax.ShapeDtypeStruct((NW, BPW), jnp.float32),
            jax.ShapeDtypeStruct((NW, BPW), jnp.float32),
        ),
        mesh=mesh,
        scratch_types=(
            pltpu.VMEM((NBLK, BC), jnp.int32),
            pltpu.VMEM((NBLK, BC), jnp.int32),
            pltpu.VMEM((NBLK * NCHUNK, CHUNK), jnp.int32),
            pltpu.VMEM((BPW,), jnp.int32),
            pltpu.VMEM((BPW,), jnp.int32),
            pltpu.VMEM((BPW * NEG,), jnp.int32),
            pltpu.VMEM((BC, 128), jnp.float32),
            pltpu.VMEM((BC, 128), jnp.float32),
            pltpu.VMEM((NROWS_BLK, 128), jnp.float32),
            pltpu.VMEM((BPW,), jnp.float32),
            pltpu.VMEM((BPW,), jnp.float32),
            pltpu.SemaphoreType.DMA,
        ),
        compiler_params=pltpu.CompilerParams(needs_layout_passes=False),
    )


def _loss_body(pos_ref, negl_ref, out_ref):
    def logsig(x):
        return jnp.minimum(x, 0.0) - jnp.log1p(jnp.exp(-jnp.abs(x)))

    total = jnp.sum(logsig(pos_ref[...])) + jnp.sum(logsig(negl_ref[...]))
    out_ref[0, 0] = -total / BATCH


def _split_idx(v):
    q = v // S
    return (v - q * S).astype(jnp.int32), (q * 32).astype(jnp.int32)


@jax.jit
def kernel(inputs, targets, neg_samples, W_w, W_u):
    wp_w, wp_u = _pack_tables(W_w, W_u)

    isup, icb = _split_idx(inputs.astype(jnp.int32).reshape(BATCH))
    tsup, tcb = _split_idx(targets.astype(jnp.int32).reshape(BATCH))
    nsup, ncb = _split_idx(neg_samples.astype(jnp.int32))

    isup_h = isup.reshape(NW, NBLK, BC)
    tsup_h = tsup.reshape(NW, NBLK, BC)
    nsup_h = nsup.reshape(NW, NBLK * NCHUNK, CHUNK)
    icb_h = icb.reshape(NW, BPW)
    tcb_h = tcb.reshape(NW, BPW)
    ncb_h = ncb.reshape(NW, BPW, NEG).transpose(0, 2, 1).reshape(NW, BPW * NEG)

    pos, negl = _make_sc_kernel()(
        wp_w, wp_u, isup_h, tsup_h, nsup_h, icb_h, tcb_h, ncb_h)

    loss = pl.pallas_call(
        _loss_body,
        out_shape=jax.ShapeDtypeStruct((1, 1), jnp.float32),
        out_specs=pl.BlockSpec(memory_space=pltpu.SMEM),
    )(pos.reshape(128, 128), negl.reshape(128, 128))
    return loss[0, 0]


# speedup vs baseline: 3.9512x; 1.3256x over previous
import functools

import jax
import jax.numpy as jnp
from jax import lax
from jax.experimental import pallas as pl
from jax.experimental.pallas import tpu as pltpu
from jax.experimental.pallas import tpu_sc as plsc

VOCAB = 1000000
EMBED = 32
BATCH = 16384
NEG = 20

NUM_CORES = 2
NUM_SUBCORES = 16
NW = NUM_CORES * NUM_SUBCORES
BPW = BATCH // NW
BC = 16
NBLK = BPW // BC
NROWS_BLK = BC * NEG
NCHUNK = 5
CHUNK = NROWS_BLK // NCHUNK

S = 250880
GW = 5120
SBG = S // GW
CLAMP_J = (VOCAB + GW - 1) // GW - 1


def _pack_body(w0, w1, w2, w3, u0, u1, u2, u3, ow, ou):
    ow[...] = jnp.concatenate(
        [w0[...], w1[...], w2[...], w3[...]], axis=0).T
    ou[...] = jnp.concatenate(
        [u0[...], u1[...], u2[...], u3[...]], axis=0).T


def _pack_tables(W_w, W_u):
    wt = W_w.T
    ut = W_u.T

    def in_spec(q):
        return pl.BlockSpec(
            (EMBED, GW),
            lambda i, q=q: (0, jnp.minimum(q * SBG + i, CLAMP_J)))

    out_spec = pl.BlockSpec((GW, 128), lambda i: (i, 0))
    return pl.pallas_call(
        _pack_body,
        grid=(SBG,),
        in_specs=[in_spec(q) for q in range(4)] * 2,
        out_specs=[out_spec, out_spec],
        out_shape=[jax.ShapeDtypeStruct((S, 128), jnp.float32)] * 2,
    )(wt, wt, wt, wt, ut, ut, ut, ut)


def _sc_body(wp_w, wp_u, isup_h, tsup_h, nsup_h, icb_h, tcb_h, ncb_h,
             pos_h, negl_h,
             isup_v, tsup_v, nsup_v, icb_v, tcb_v, ncb_v,
             emb0, ctx0, nrow0, emb1, ctx1, nrow1, pos_v, negl_v, sem):
    cid = lax.axis_index("c")
    sid = lax.axis_index("s")
    wid = sid * NUM_CORES + cid

    pltpu.sync_copy(isup_h.at[wid], isup_v)
    pltpu.sync_copy(tsup_h.at[wid], tsup_v)
    pltpu.sync_copy(nsup_h.at[wid], nsup_v)
    pltpu.sync_copy(icb_h.at[wid], icb_v)
    pltpu.sync_copy(tcb_h.at[wid], tcb_v)
    pltpu.sync_copy(ncb_h.at[wid], ncb_v)

    iota = lax.iota(jnp.int32, 16)
    nrows0 = iota * NEG
    bufs = ((emb0, ctx0, nrow0), (emb1, ctx1, nrow1))

    def issue(g, emb_b, ctx_b, nrow_b):
        pltpu.async_copy(wp_w.at[isup_v.at[pl.ds(g * BC, BC)]], emb_b, sem)
        pltpu.async_copy(wp_u.at[tsup_v.at[pl.ds(g * BC, BC)]], ctx_b, sem)
        for c in range(NCHUNK):
            pltpu.async_copy(
                wp_u.at[nsup_v.at[pl.ds((g * NCHUNK + c) * CHUNK, CHUNK)]],
                nrow_b.at[pl.ds(c * CHUNK, CHUNK)], sem)

    def drain(emb_b, ctx_b, nrow_b):
        pltpu.make_async_copy(wp_w.at[pl.ds(0, BC)], emb_b, sem).wait()
        pltpu.make_async_copy(wp_u.at[pl.ds(0, BC)], ctx_b, sem).wait()
        for c in range(NCHUNK):
            pltpu.make_async_copy(wp_u.at[pl.ds(0, CHUNK)],
                                  nrow_b.at[pl.ds(c * CHUNK, CHUNK)],
                                  sem).wait()

    def compute(blk, emb_b, ctx_b, nrow_b):
        cb_e = icb_v[pl.ds(blk * BC, 16)]
        cb_c = tcb_v[pl.ds(blk * BC, 16)]
        zero = jnp.zeros((16,), jnp.float32)

        @pl.loop(0, EMBED, init_carry=(zero, zero))
        def _dim(d, carry):
            acc_p, acc_n = carry
            e = plsc.load_gather(emb_b, [iota, cb_e + d])
            c = plsc.load_gather(ctx_b, [iota, cb_c + d])
            s = jnp.zeros((16,), jnp.float32)
            for k in range(NEG):
                cb_nk = ncb_v[pl.ds(k * BPW + blk * BC, 16)]
                s = s + plsc.load_gather(nrow_b, [nrows0 + k, cb_nk + d])
            return acc_p + e * c, acc_n + e * s

        acc_p, acc_n = _dim
        pos_v[pl.ds(blk * BC, 16)] = acc_p
        negl_v[pl.ds(blk * BC, 16)] = -acc_n

    issue(0, *bufs[0])

    @pl.loop(0, NBLK, step=2)
    def _pair(blk):
        for b in (0, 1):
            g = blk + b

            @pl.when(g + 1 < NBLK)
            def _():
                issue(g + 1, *bufs[1 - b])

            drain(*bufs[b])
            compute(g, *bufs[b])

    pltpu.sync_copy(pos_v, pos_h.at[wid])
    pltpu.sync_copy(negl_v, negl_h.at[wid])


def _make_sc_kernel():
    mesh = plsc.VectorSubcoreMesh(core_axis_name="c", subcore_axis_name="s")
    return pl.kernel(
        _sc_body,
        out_type=(
            jax.ShapeDtypeStruct((NW, BPW), jnp.float32),
            jax.ShapeDtypeStruct((NW, BPW), jnp.float32),
        ),
        mesh=mesh,
        scratch_types=(
            pltpu.VMEM((BPW,), jnp.int32),
            pltpu.VMEM((BPW,), jnp.int32),
            pltpu.VMEM((BPW * NEG,), jnp.int32),
            pltpu.VMEM((BPW,), jnp.int32),
            pltpu.VMEM((BPW,), jnp.int32),
            pltpu.VMEM((BPW * NEG,), jnp.int32),
            pltpu.VMEM((BC, 128), jnp.float32),
            pltpu.VMEM((BC, 128), jnp.float32),
            pltpu.VMEM((NROWS_BLK, 128), jnp.float32),
            pltpu.VMEM((BC, 128), jnp.float32),
            pltpu.VMEM((BC, 128), jnp.float32),
            pltpu.VMEM((NROWS_BLK, 128), jnp.float32),
            pltpu.VMEM((BPW,), jnp.float32),
            pltpu.VMEM((BPW,), jnp.float32),
            pltpu.SemaphoreType.DMA,
        ),
        compiler_params=pltpu.CompilerParams(needs_layout_passes=False),
    )


def _loss_body(pos_ref, negl_ref, out_ref):
    def logsig(x):
        return jnp.minimum(x, 0.0) - jnp.log1p(jnp.exp(-jnp.abs(x)))

    total = jnp.sum(logsig(pos_ref[...])) + jnp.sum(logsig(negl_ref[...]))
    out_ref[0, 0] = -total / BATCH


def _split_idx(v):
    q = v // S
    return (v - q * S).astype(jnp.int32), (q * 32).astype(jnp.int32)


@jax.jit
def kernel(inputs, targets, neg_samples, W_w, W_u):
    wp_w, wp_u = _pack_tables(W_w, W_u)

    isup, icb = _split_idx(inputs.astype(jnp.int32).reshape(BATCH))
    tsup, tcb = _split_idx(targets.astype(jnp.int32).reshape(BATCH))
    nsup, ncb = _split_idx(neg_samples.astype(jnp.int32))

    isup_h = isup.reshape(NW, BPW)
    tsup_h = tsup.reshape(NW, BPW)
    nsup_h = nsup.reshape(NW, BPW * NEG)
    icb_h = icb.reshape(NW, BPW)
    tcb_h = tcb.reshape(NW, BPW)
    ncb_h = ncb.reshape(NW, BPW, NEG).transpose(0, 2, 1).reshape(NW, BPW * NEG)

    pos, negl = _make_sc_kernel()(
        wp_w, wp_u, isup_h, tsup_h, nsup_h, icb_h, tcb_h, ncb_h)

    loss = pl.pallas_call(
        _loss_body,
        out_shape=jax.ShapeDtypeStruct((1, 1), jnp.float32),
        out_specs=pl.BlockSpec(memory_space=pltpu.SMEM),
    )(pos.reshape(128, 128), negl.reshape(128, 128))
    return loss[0, 0]
